# baseline (speedup 1.0000x reference)
import functools

import jax
import jax.numpy as jnp
from jax import lax
from jax.experimental import pallas as pl
from jax.experimental.pallas import tpu as pltpu
from jax.experimental.pallas import tpu_sc as plsc

BATCH = 16384
EMBED_DIM = 64

_info = plsc.get_sparse_core_info()
_NC = _info.num_cores
_NS = _info.num_subcores
_NW = _NC * _NS
_BPW = BATCH // _NW
_CHUNK = 128
_NCHUNK = _BPW // _CHUNK

_mesh = plsc.VectorSubcoreMesh(core_axis_name="c", subcore_axis_name="s")


@functools.partial(
    pl.kernel,
    mesh=_mesh,
    out_type=jax.ShapeDtypeStruct((BATCH // 2, 128), jnp.float32),
    scratch_types=[
        pltpu.VMEM((_BPW,), jnp.int32),
        pltpu.VMEM((_BPW,), jnp.int32),
        pltpu.VMEM((_BPW,), jnp.int32),
        pltpu.VMEM((_BPW, 128), jnp.float32),
        pltpu.VMEM((_BPW // 2, 128), jnp.float32),
        pltpu.SemaphoreType.DMA,
    ],
    compiler_params=pltpu.CompilerParams(
        use_tc_tiling_on_sc=False, needs_layout_passes=False),
)
def _gather_packed(wp_hbm, idx_hbm, out_hbm,
                   idx_v, pid_v, off_v, rows_v, out_v, sem_g):
    wid = lax.axis_index("s") * _NC + lax.axis_index("c")
    base = wid * _BPW
    pltpu.sync_copy(idx_hbm.at[pl.ds(base, _BPW)], idx_v)
    for k in range(_BPW // 16):
        s = pl.ds(k * 16, 16)
        v = idx_v[s]
        pid_v[s] = jax.lax.shift_right_logical(v, 1)
        off_v[s] = (v & 1) * 64
    handles = [
        pltpu.async_copy(
            wp_hbm.at[pid_v.at[pl.ds(j * _CHUNK, _CHUNK)]],
            rows_v.at[pl.ds(j * _CHUNK, _CHUNK)],
            sem_g,
        )
        for j in range(_NCHUNK)
    ]
    for h in handles:
        h.wait()

    lanes = lax.iota(jnp.int32, 16)

    def compact(g, _):
        r0 = g * 16
        offs = off_v[pl.ds(r0, 16)]
        for l in range(16):
            off = jnp.sum(jnp.where(lanes == l, offs, 0))
            q = g * 8 + l // 2
            h0 = (l % 2) * 64
            for k in range(4):
                out_v[q, pl.ds(h0 + k * 16, 16)] = (
                    rows_v[r0 + l, pl.ds(off + k * 16, 16)])
        return 0

    lax.fori_loop(0, _BPW // 16, compact, 0)
    pltpu.sync_copy(out_v, out_hbm.at[pl.ds(wid * (_BPW // 2), _BPW // 2)])


def kernel(x, weight):
    wp = weight.reshape(500000, 128)
    xi = x.astype(jnp.int32)
    outp = _gather_packed(wp, xi)
    return outp.reshape(BATCH, EMBED_DIM)


# speedup vs baseline: 1.1417x; 1.1417x over previous
import functools

import jax
import jax.numpy as jnp
from jax import lax
from jax.experimental import pallas as pl
from jax.experimental.pallas import tpu as pltpu
from jax.experimental.pallas import tpu_sc as plsc

BATCH = 16384
EMBED_DIM = 64

_info = plsc.get_sparse_core_info()
_NC = _info.num_cores
_NS = _info.num_subcores
_NW = _NC * _NS
_BPW = BATCH // _NW
_CHUNK = 128
_NCHUNK = _BPW // _CHUNK

_mesh = plsc.VectorSubcoreMesh(core_axis_name="c", subcore_axis_name="s")


@functools.partial(
    pl.kernel,
    mesh=_mesh,
    out_type=jax.ShapeDtypeStruct((BATCH, 128), jnp.float32),
    scratch_types=[
        pltpu.VMEM((_BPW,), jnp.int32),
        pltpu.VMEM((_BPW, 128), jnp.float32),
        pltpu.SemaphoreType.DMA,
    ],
    compiler_params=pltpu.CompilerParams(use_tc_tiling_on_sc=False),
)
def _gather_rows(w_hbm, idx_hbm, out_hbm, idx_v, rows_v, sem):
    wid = lax.axis_index("s") * _NC + lax.axis_index("c")
    base = wid * _BPW
    pltpu.sync_copy(idx_hbm.at[pl.ds(base, _BPW)], idx_v)
    handles = [
        pltpu.async_copy(
            w_hbm.at[idx_v.at[pl.ds(j * _CHUNK, _CHUNK)]],
            rows_v.at[pl.ds(j * _CHUNK, _CHUNK)],
            sem,
        )
        for j in range(_NCHUNK)
    ]
    for h in handles:
        h.wait()
    pltpu.sync_copy(rows_v, out_hbm.at[pl.ds(base, _BPW)])


def kernel(x, weight):
    wpad = jnp.pad(weight, ((0, 0), (0, 128 - EMBED_DIM)))
    out128 = _gather_rows(wpad, x.astype(jnp.int32))
    return out128[:, :EMBED_DIM]
